# baseline (speedup 1.0000x reference)
import jax
import jax.numpy as jnp
from jax import lax
from jax.experimental import pallas as pl
from jax.experimental.pallas import tpu as pltpu
from jax.experimental.pallas import tpu_sc as plsc

N_NODES = 10000
N_EDGES = 320000
D = 128
ED = 16
LN_EPS = 1e-5

NC = 2
NS = 16
NW = NC * NS
CH = 128
NCH = 80
E_PAD = NW * NCH * CH
N_PAD = 10240


def _sc_mesh():
    return plsc.VectorSubcoreMesh(
        core_axis_name="c", subcore_axis_name="s",
        num_cores=NC, num_subcores=NS,
    )


def _sc_gather_body(x_hbm, src_hbm, out_hbm, idx_v, buf0, buf1,
                    gsem0, gsem1, wsem0, wsem1):
    c = lax.axis_index("c")
    s = lax.axis_index("s")
    wid = s * NC + c
    pltpu.sync_copy(src_hbm.at[wid], idx_v)

    def body(j, carry):
        pltpu.async_copy(x_hbm.at[idx_v.at[j]], buf0, gsem0).wait()
        pltpu.sync_copy(buf0, out_hbm.at[wid, j])
        return carry

    lax.fori_loop(0, NCH, body, 0)


def _make_sc_gather():
    return pl.kernel(
        _sc_gather_body,
        out_type=jax.ShapeDtypeStruct((NW, NCH, CH, D), jnp.float32),
        mesh=_sc_mesh(),
        scratch_types=[
            pltpu.VMEM((NCH, CH), jnp.int32),
            pltpu.VMEM((CH, D), jnp.float32),
            pltpu.VMEM((CH, D), jnp.float32),
            pltpu.SemaphoreType.DMA,
            pltpu.SemaphoreType.DMA,
            pltpu.SemaphoreType.DMA,
            pltpu.SemaphoreType.DMA,
        ],
    )


def _sc_scatter_body(msg_hbm, dst_hbm, out_hbm, idx_v, buf0, buf1, acc,
                     rsem0, rsem1, asem0, asem1):
    c = lax.axis_index("c")
    s = lax.axis_index("s")
    wid = s * NC + c
    rows_per_s = N_PAD // NS

    def zero_buf(i, carry):
        buf0[i // 8, pl.ds((i % 8) * 16, 16)] = jnp.zeros((16,), jnp.float32)
        return carry

    lax.fori_loop(0, (CH * D) // 16, zero_buf, 0)

    def zero_acc(k, carry):
        pltpu.sync_copy(buf0, acc.at[pl.ds(s * rows_per_s + k * CH, CH)])
        return carry

    lax.fori_loop(0, rows_per_s // CH, zero_acc, 0)
    plsc.subcore_barrier()

    pltpu.sync_copy(dst_hbm.at[wid], idx_v)

    pltpu.async_copy(msg_hbm.at[wid, 0], buf0, rsem0)

    def pair(k, carry):
        a = 2 * k
        b = a + 1
        pltpu.make_async_copy(msg_hbm.at[wid, a], buf0, rsem0).wait()

        @pl.when(k > 0)
        def _():
            pltpu.make_async_copy(buf1, acc.at[idx_v.at[b - 2]], asem1).wait()

        pltpu.async_copy(msg_hbm.at[wid, b], buf1, rsem1)
        pltpu.async_copy(buf0, acc.at[idx_v.at[a]], asem0, add=True)
        pltpu.make_async_copy(msg_hbm.at[wid, b], buf1, rsem1).wait()
        pltpu.make_async_copy(buf0, acc.at[idx_v.at[a]], asem0).wait()

        @pl.when(k < NCH // 2 - 1)
        def _():
            pltpu.async_copy(msg_hbm.at[wid, a + 2], buf0, rsem0)

        pltpu.async_copy(buf1, acc.at[idx_v.at[b]], asem1, add=True)
        return carry

    lax.fori_loop(0, NCH // 2, pair, 0)
    pltpu.make_async_copy(buf1, acc.at[idx_v.at[NCH - 1]], asem1).wait()
    plsc.subcore_barrier()
    pltpu.sync_copy(
        acc.at[pl.ds(s * rows_per_s, rows_per_s)],
        out_hbm.at[c, pl.ds(s * rows_per_s, rows_per_s)],
    )


def _make_sc_scatter():
    return pl.kernel(
        _sc_scatter_body,
        out_type=jax.ShapeDtypeStruct((NC, N_PAD, D), jnp.float32),
        mesh=_sc_mesh(),
        scratch_types=[
            pltpu.VMEM((NCH, CH), jnp.int32),
            pltpu.VMEM((CH, D), jnp.float32),
            pltpu.VMEM((CH, D), jnp.float32),
            pltpu.VMEM_SHARED((N_PAD, D), jnp.float32),
            pltpu.SemaphoreType.DMA,
            pltpu.SemaphoreType.DMA,
            pltpu.SemaphoreType.DMA,
            pltpu.SemaphoreType.DMA,
        ],
    )


E_BLK = 4096


def _msg_body(ea_ref, xg_ref, We_ref, be_ref, Wm_ref, bm_ref, out_ref):
    e = jnp.dot(ea_ref[...], We_ref[...], preferred_element_type=jnp.float32)
    e = e + be_ref[...]
    e = e * jax.nn.sigmoid(e)
    t = jnp.maximum(xg_ref[...] + e, 0.0)
    m = jnp.dot(t, Wm_ref[...], preferred_element_type=jnp.float32) + bm_ref[...]
    out_ref[...] = m * jax.nn.sigmoid(m)


def _msg_call(ea, xg, We, be, Wm, bm):
    grid = (E_PAD // E_BLK,)
    return pl.pallas_call(
        _msg_body,
        grid=grid,
        in_specs=[
            pl.BlockSpec((E_BLK, ED), lambda i: (i, 0)),
            pl.BlockSpec((E_BLK, D), lambda i: (i, 0)),
            pl.BlockSpec((ED, D), lambda i: (0, 0)),
            pl.BlockSpec((1, D), lambda i: (0, 0)),
            pl.BlockSpec((D, D), lambda i: (0, 0)),
            pl.BlockSpec((1, D), lambda i: (0, 0)),
        ],
        out_specs=pl.BlockSpec((E_BLK, D), lambda i: (i, 0)),
        out_shape=jax.ShapeDtypeStruct((E_PAD, D), jnp.float32),
    )(ea, xg, We, be, Wm, bm)


def _final_body(aggA_ref, aggB_ref, x_ref, n2g_col_ref, n2g_row_ref, eps_ref,
                gw_ref, gb_ref, out_ref):
    x = x_ref[...]
    h = aggA_ref[...] + aggB_ref[...] + (1.0 + eps_ref[0, 0]) * x

    gid_cols = lax.broadcasted_iota(jnp.int32, (N_NODES, D), 1)
    oh = (gid_cols == n2g_col_ref[...]).astype(jnp.float32)
    gid_rows = lax.broadcasted_iota(jnp.int32, (D, N_NODES), 0)
    ohT = (gid_rows == n2g_row_ref[...]).astype(jnp.float32)

    cnt = jnp.sum(ohT, axis=1, keepdims=True)
    norm = jnp.maximum(cnt, 1.0) * jnp.float32(D)

    s1 = jnp.dot(ohT, h, preferred_element_type=jnp.float32)
    mean_g = jnp.sum(s1, axis=1, keepdims=True) / norm
    mean_n = jnp.dot(oh, mean_g, preferred_element_type=jnp.float32)
    xc = h - mean_n
    s2 = jnp.dot(ohT, xc * xc, preferred_element_type=jnp.float32)
    var_g = jnp.sum(s2, axis=1, keepdims=True) / norm
    rstd_g = lax.rsqrt(var_g + jnp.float32(LN_EPS))
    rstd_n = jnp.dot(oh, rstd_g, preferred_element_type=jnp.float32)

    y = xc * rstd_n * gw_ref[...] + gb_ref[...] + x
    out_ref[...] = jnp.maximum(y, 0.0)


def _final_call(aggA, aggB, x, n2g_col, n2g_row, eps, gw, gb):
    return pl.pallas_call(
        _final_body,
        out_shape=jax.ShapeDtypeStruct((N_NODES, D), jnp.float32),
    )(aggA, aggB, x, n2g_col, n2g_row, eps, gw, gb)


def kernel(x, edge_index, edge_attr, node2graph,
           We1, be1, Wm1, bm1, eps1, gn1_w, gn1_b,
           We2, be2, Wm2, bm2, eps2, gn2_w, gn2_b):
    n_pad_edges = E_PAD - N_EDGES
    src = jnp.concatenate(
        [edge_index[0], jnp.zeros((n_pad_edges,), jnp.int32)]
    ).reshape(NW, NCH, CH)
    dst = jnp.concatenate(
        [edge_index[1], jnp.full((n_pad_edges,), N_NODES, jnp.int32)]
    ).reshape(NW, NCH, CH)
    ea = jnp.concatenate(
        [edge_attr, jnp.zeros((n_pad_edges, ED), jnp.float32)], axis=0
    )

    xg = _make_sc_gather()(x, src).reshape(E_PAD, D)
    msg = _msg_call(ea, xg, We2, be2.reshape(1, D), Wm2, bm2.reshape(1, D))
    agg2 = _make_sc_scatter()(msg.reshape(NW, NCH, CH, D), dst)
    out = _final_call(
        agg2[0, :N_NODES], agg2[1, :N_NODES], x,
        node2graph.reshape(N_NODES, 1), node2graph.reshape(1, N_NODES),
        eps2.reshape(1, 1), gn2_w.reshape(1, D), gn2_b.reshape(1, D),
    )
    return out


# speedup vs baseline: 1.2932x; 1.2932x over previous
import jax
import jax.numpy as jnp
from jax import lax
from jax.experimental import pallas as pl
from jax.experimental.pallas import tpu as pltpu
from jax.experimental.pallas import tpu_sc as plsc

N_NODES = 10000
N_EDGES = 320000
D = 128
ED = 16
LN_EPS = 1e-5

NC = 2
NS = 16
NW = NC * NS
CH = 128
NCH = 79
E_PAD = NW * NCH * CH
N_PAD = 10240


def _sc_mesh():
    return plsc.VectorSubcoreMesh(
        core_axis_name="c", subcore_axis_name="s",
        num_cores=NC, num_subcores=NS,
    )


def _sc_gather_body(x_hbm, src_hbm, out_hbm, idx_v, buf0, buf1,
                    gsem0, gsem1, wsem0, wsem1):
    c = lax.axis_index("c")
    s = lax.axis_index("s")
    wid = s * NC + c
    pltpu.sync_copy(src_hbm.at[wid], idx_v)

    def body(j, carry):
        pltpu.async_copy(x_hbm.at[idx_v.at[j]], buf0, gsem0).wait()
        pltpu.sync_copy(buf0, out_hbm.at[wid, j])
        return carry

    lax.fori_loop(0, NCH, body, 0)


def _make_sc_gather():
    return pl.kernel(
        _sc_gather_body,
        out_type=jax.ShapeDtypeStruct((NW, NCH, CH, D), jnp.float32),
        mesh=_sc_mesh(),
        scratch_types=[
            pltpu.VMEM((NCH, CH), jnp.int32),
            pltpu.VMEM((CH, D), jnp.float32),
            pltpu.VMEM((CH, D), jnp.float32),
            pltpu.SemaphoreType.DMA,
            pltpu.SemaphoreType.DMA,
            pltpu.SemaphoreType.DMA,
            pltpu.SemaphoreType.DMA,
        ],
    )


def _sc_scatter_body(msg_hbm, dst_hbm, out_hbm, idx_v, buf0, buf1, acc,
                     rsem0, rsem1, asem0, asem1):
    c = lax.axis_index("c")
    s = lax.axis_index("s")
    wid = s * NC + c
    rows_per_s = N_PAD // NS

    def zero_buf(i, carry):
        buf0[i // 8, pl.ds((i % 8) * 16, 16)] = jnp.zeros((16,), jnp.float32)
        return carry

    lax.fori_loop(0, (CH * D) // 16, zero_buf, 0)

    def zero_acc(k, carry):
        pltpu.sync_copy(buf0, acc.at[pl.ds(s * rows_per_s + k * CH, CH)])
        return carry

    lax.fori_loop(0, rows_per_s // CH, zero_acc, 0)
    plsc.subcore_barrier()

    pltpu.sync_copy(dst_hbm.at[wid], idx_v)

    pltpu.async_copy(msg_hbm.at[wid, 0], buf0, rsem0)

    def pair(k, carry):
        a = 2 * k
        b = a + 1
        pltpu.make_async_copy(msg_hbm.at[wid, a], buf0, rsem0).wait()

        @pl.when(k > 0)
        def _():
            pltpu.make_async_copy(buf1, acc.at[idx_v.at[b - 2]], asem1).wait()

        pltpu.async_copy(msg_hbm.at[wid, b], buf1, rsem1)
        pltpu.async_copy(buf0, acc.at[idx_v.at[a]], asem0, add=True)
        pltpu.make_async_copy(msg_hbm.at[wid, b], buf1, rsem1).wait()
        pltpu.make_async_copy(buf0, acc.at[idx_v.at[a]], asem0).wait()

        @pl.when(k < NCH // 2 - 1)
        def _():
            pltpu.async_copy(msg_hbm.at[wid, a + 2], buf0, rsem0)

        pltpu.async_copy(buf1, acc.at[idx_v.at[b]], asem1, add=True)
        return carry

    lax.fori_loop(0, NCH // 2, pair, 0)
    pltpu.async_copy(msg_hbm.at[wid, NCH - 1], buf0, rsem0).wait()
    pltpu.make_async_copy(buf1, acc.at[idx_v.at[NCH - 2]], asem1).wait()
    pltpu.sync_copy(buf0, acc.at[idx_v.at[NCH - 1]], add=True)
    plsc.subcore_barrier()
    pltpu.sync_copy(
        acc.at[pl.ds(s * rows_per_s, rows_per_s)],
        out_hbm.at[c, pl.ds(s * rows_per_s, rows_per_s)],
    )


def _make_sc_scatter():
    return pl.kernel(
        _sc_scatter_body,
        out_type=jax.ShapeDtypeStruct((NC, N_PAD, D), jnp.float32),
        mesh=_sc_mesh(),
        scratch_types=[
            pltpu.VMEM((NCH, CH), jnp.int32),
            pltpu.VMEM((CH, D), jnp.float32),
            pltpu.VMEM((CH, D), jnp.float32),
            pltpu.VMEM_SHARED((N_PAD, D), jnp.float32),
            pltpu.SemaphoreType.DMA,
            pltpu.SemaphoreType.DMA,
            pltpu.SemaphoreType.DMA,
            pltpu.SemaphoreType.DMA,
        ],
    )


E_BLK = 4096


def _msg_body(ea_ref, xg_ref, We_ref, be_ref, Wm_ref, bm_ref, out_ref):
    e = jnp.dot(ea_ref[...], We_ref[...], preferred_element_type=jnp.float32)
    e = e + be_ref[...]
    e = e * jax.nn.sigmoid(e)
    t = jnp.maximum(xg_ref[...] + e, 0.0)
    m = jnp.dot(t, Wm_ref[...], preferred_element_type=jnp.float32) + bm_ref[...]
    out_ref[...] = m * jax.nn.sigmoid(m)


def _msg_call(ea, xg, We, be, Wm, bm):
    grid = (E_PAD // E_BLK,)
    return pl.pallas_call(
        _msg_body,
        grid=grid,
        in_specs=[
            pl.BlockSpec((E_BLK, ED), lambda i: (i, 0)),
            pl.BlockSpec((E_BLK, D), lambda i: (i, 0)),
            pl.BlockSpec((ED, D), lambda i: (0, 0)),
            pl.BlockSpec((1, D), lambda i: (0, 0)),
            pl.BlockSpec((D, D), lambda i: (0, 0)),
            pl.BlockSpec((1, D), lambda i: (0, 0)),
        ],
        out_specs=pl.BlockSpec((E_BLK, D), lambda i: (i, 0)),
        out_shape=jax.ShapeDtypeStruct((E_PAD, D), jnp.float32),
    )(ea, xg, We, be, Wm, bm)


def _final_body(aggA_ref, aggB_ref, x_ref, n2g_col_ref, n2g_row_ref, eps_ref,
                gw_ref, gb_ref, out_ref):
    x = x_ref[...]
    h = aggA_ref[...] + aggB_ref[...] + (1.0 + eps_ref[0, 0]) * x

    gid_cols = lax.broadcasted_iota(jnp.int32, (N_NODES, D), 1)
    oh = (gid_cols == n2g_col_ref[...]).astype(jnp.float32)
    gid_rows = lax.broadcasted_iota(jnp.int32, (D, N_NODES), 0)
    ohT = (gid_rows == n2g_row_ref[...]).astype(jnp.float32)

    cnt = jnp.sum(ohT, axis=1, keepdims=True)
    norm = jnp.maximum(cnt, 1.0) * jnp.float32(D)

    s1 = jnp.dot(ohT, h, preferred_element_type=jnp.float32)
    mean_g = jnp.sum(s1, axis=1, keepdims=True) / norm
    mean_n = jnp.dot(oh, mean_g, preferred_element_type=jnp.float32)
    xc = h - mean_n
    s2 = jnp.dot(ohT, xc * xc, preferred_element_type=jnp.float32)
    var_g = jnp.sum(s2, axis=1, keepdims=True) / norm
    rstd_g = lax.rsqrt(var_g + jnp.float32(LN_EPS))
    rstd_n = jnp.dot(oh, rstd_g, preferred_element_type=jnp.float32)

    y = xc * rstd_n * gw_ref[...] + gb_ref[...] + x
    out_ref[...] = jnp.maximum(y, 0.0)


def _final_call(aggA, aggB, x, n2g_col, n2g_row, eps, gw, gb):
    return pl.pallas_call(
        _final_body,
        out_shape=jax.ShapeDtypeStruct((N_NODES, D), jnp.float32),
    )(aggA, aggB, x, n2g_col, n2g_row, eps, gw, gb)


def kernel(x, edge_index, edge_attr, node2graph,
           We1, be1, Wm1, bm1, eps1, gn1_w, gn1_b,
           We2, be2, Wm2, bm2, eps2, gn2_w, gn2_b):
    n_pad_edges = E_PAD - N_EDGES
    src = jnp.concatenate(
        [edge_index[0], jnp.zeros((n_pad_edges,), jnp.int32)]
    ).reshape(NW, NCH, CH)
    dst = jnp.concatenate(
        [edge_index[1], jnp.full((n_pad_edges,), N_NODES, jnp.int32)]
    ).reshape(NW, NCH, CH)
    ea = jnp.concatenate(
        [edge_attr, jnp.zeros((n_pad_edges, ED), jnp.float32)], axis=0
    )

    xg = _make_sc_gather()(x, src).reshape(E_PAD, D)
    msg = _msg_call(ea, xg, We2, be2.reshape(1, D), Wm2, bm2.reshape(1, D))
    agg2 = _make_sc_scatter()(msg.reshape(NW, NCH, CH, D), dst)
    out = _final_call(
        agg2[0, :N_NODES], agg2[1, :N_NODES], x,
        node2graph.reshape(N_NODES, 1), node2graph.reshape(1, N_NODES),
        eps2.reshape(1, 1), gn2_w.reshape(1, D), gn2_b.reshape(1, D),
    )
    return out


# speedup vs baseline: 1.9812x; 1.5320x over previous
import jax
import jax.numpy as jnp
from jax import lax
from jax.experimental import pallas as pl
from jax.experimental.pallas import tpu as pltpu
from jax.experimental.pallas import tpu_sc as plsc

N_NODES = 10000
N_EDGES = 320000
D = 128
ED = 16
LN_EPS = 1e-5

NC = 2
NS = 16
NW = NC * NS
CH = 128
NCH = 79
E_PAD = NW * NCH * CH
N_PAD = 10240


def _sc_mesh():
    return plsc.VectorSubcoreMesh(
        core_axis_name="c", subcore_axis_name="s",
        num_cores=NC, num_subcores=NS,
    )


def _sc_gather_body(x_hbm, src_hbm, out_hbm, idx_v, buf0, buf1, xs,
                    gsem0, gsem1, wsem0, wsem1):
    c = lax.axis_index("c")
    s = lax.axis_index("s")
    wid = s * NC + c
    stage_rows = N_PAD // NS

    pltpu.sync_copy(
        x_hbm.at[pl.ds(s * stage_rows, stage_rows)],
        xs.at[pl.ds(s * stage_rows, stage_rows)],
    )
    pltpu.sync_copy(src_hbm.at[wid], idx_v)
    plsc.subcore_barrier()

    pltpu.async_copy(xs.at[idx_v.at[0]], buf0, gsem0)

    def pair(k, carry):
        a = 2 * k
        b = a + 1
        pltpu.make_async_copy(xs.at[idx_v.at[a]], buf0, gsem0).wait()

        @pl.when(k > 0)
        def _():
            pltpu.make_async_copy(buf1, out_hbm.at[wid, b - 2], wsem1).wait()

        pltpu.async_copy(xs.at[idx_v.at[b]], buf1, gsem1)
        pltpu.async_copy(buf0, out_hbm.at[wid, a], wsem0)
        pltpu.make_async_copy(xs.at[idx_v.at[b]], buf1, gsem1).wait()
        pltpu.make_async_copy(buf0, out_hbm.at[wid, a], wsem0).wait()

        @pl.when(k < NCH // 2 - 1)
        def _():
            pltpu.async_copy(xs.at[idx_v.at[a + 2]], buf0, gsem0)

        pltpu.async_copy(buf1, out_hbm.at[wid, b], wsem1)
        return carry

    lax.fori_loop(0, NCH // 2, pair, 0)
    pltpu.async_copy(xs.at[idx_v.at[NCH - 1]], buf0, gsem0).wait()
    pltpu.make_async_copy(buf1, out_hbm.at[wid, NCH - 2], wsem1).wait()
    pltpu.sync_copy(buf0, out_hbm.at[wid, NCH - 1])


def _make_sc_gather():
    return pl.kernel(
        _sc_gather_body,
        out_type=jax.ShapeDtypeStruct((NW, NCH, CH, D), jnp.float32),
        mesh=_sc_mesh(),
        scratch_types=[
            pltpu.VMEM((NCH, CH), jnp.int32),
            pltpu.VMEM((CH, D), jnp.float32),
            pltpu.VMEM((CH, D), jnp.float32),
            pltpu.VMEM_SHARED((N_PAD, D), jnp.float32),
            pltpu.SemaphoreType.DMA,
            pltpu.SemaphoreType.DMA,
            pltpu.SemaphoreType.DMA,
            pltpu.SemaphoreType.DMA,
        ],
    )


def _sc_scatter_body(msg_hbm, dst_hbm, out_hbm, idx_v, buf0, buf1, acc,
                     rsem0, rsem1, asem0, asem1):
    c = lax.axis_index("c")
    s = lax.axis_index("s")
    wid = s * NC + c
    rows_per_s = N_PAD // NS

    def zero_buf(i, carry):
        buf0[i // 8, pl.ds((i % 8) * 16, 16)] = jnp.zeros((16,), jnp.float32)
        return carry

    lax.fori_loop(0, (CH * D) // 16, zero_buf, 0)

    def zero_acc(k, carry):
        pltpu.sync_copy(buf0, acc.at[pl.ds(s * rows_per_s + k * CH, CH)])
        return carry

    lax.fori_loop(0, rows_per_s // CH, zero_acc, 0)
    plsc.subcore_barrier()

    pltpu.sync_copy(dst_hbm.at[wid], idx_v)

    pltpu.async_copy(msg_hbm.at[wid, 0], buf0, rsem0)

    def pair(k, carry):
        a = 2 * k
        b = a + 1
        pltpu.make_async_copy(msg_hbm.at[wid, a], buf0, rsem0).wait()

        @pl.when(k > 0)
        def _():
            pltpu.make_async_copy(buf1, acc.at[idx_v.at[b - 2]], asem1).wait()

        pltpu.async_copy(msg_hbm.at[wid, b], buf1, rsem1)
        pltpu.async_copy(buf0, acc.at[idx_v.at[a]], asem0, add=True)
        pltpu.make_async_copy(msg_hbm.at[wid, b], buf1, rsem1).wait()
        pltpu.make_async_copy(buf0, acc.at[idx_v.at[a]], asem0).wait()

        @pl.when(k < NCH // 2 - 1)
        def _():
            pltpu.async_copy(msg_hbm.at[wid, a + 2], buf0, rsem0)

        pltpu.async_copy(buf1, acc.at[idx_v.at[b]], asem1, add=True)
        return carry

    lax.fori_loop(0, NCH // 2, pair, 0)
    pltpu.async_copy(msg_hbm.at[wid, NCH - 1], buf0, rsem0).wait()
    pltpu.make_async_copy(buf1, acc.at[idx_v.at[NCH - 2]], asem1).wait()
    pltpu.sync_copy(buf0, acc.at[idx_v.at[NCH - 1]], add=True)
    plsc.subcore_barrier()
    pltpu.sync_copy(
        acc.at[pl.ds(s * rows_per_s, rows_per_s)],
        out_hbm.at[c, pl.ds(s * rows_per_s, rows_per_s)],
    )


def _make_sc_scatter():
    return pl.kernel(
        _sc_scatter_body,
        out_type=jax.ShapeDtypeStruct((NC, N_PAD, D), jnp.float32),
        mesh=_sc_mesh(),
        scratch_types=[
            pltpu.VMEM((NCH, CH), jnp.int32),
            pltpu.VMEM((CH, D), jnp.float32),
            pltpu.VMEM((CH, D), jnp.float32),
            pltpu.VMEM_SHARED((N_PAD, D), jnp.float32),
            pltpu.SemaphoreType.DMA,
            pltpu.SemaphoreType.DMA,
            pltpu.SemaphoreType.DMA,
            pltpu.SemaphoreType.DMA,
        ],
    )


E_BLK = 4096


def _msg_body(ea_ref, xg_ref, We_ref, be_ref, Wm_ref, bm_ref, out_ref):
    e = jnp.dot(ea_ref[...], We_ref[...], preferred_element_type=jnp.float32)
    e = e + be_ref[...]
    e = e * jax.nn.sigmoid(e)
    t = jnp.maximum(xg_ref[...] + e, 0.0)
    m = jnp.dot(t, Wm_ref[...], preferred_element_type=jnp.float32) + bm_ref[...]
    out_ref[...] = m * jax.nn.sigmoid(m)


def _msg_call(ea, xg, We, be, Wm, bm):
    grid = (E_PAD // E_BLK,)
    return pl.pallas_call(
        _msg_body,
        grid=grid,
        in_specs=[
            pl.BlockSpec((E_BLK, ED), lambda i: (i, 0)),
            pl.BlockSpec((E_BLK, D), lambda i: (i, 0)),
            pl.BlockSpec((ED, D), lambda i: (0, 0)),
            pl.BlockSpec((1, D), lambda i: (0, 0)),
            pl.BlockSpec((D, D), lambda i: (0, 0)),
            pl.BlockSpec((1, D), lambda i: (0, 0)),
        ],
        out_specs=pl.BlockSpec((E_BLK, D), lambda i: (i, 0)),
        out_shape=jax.ShapeDtypeStruct((E_PAD, D), jnp.float32),
    )(ea, xg, We, be, Wm, bm)


def _final_body(aggA_ref, aggB_ref, x_ref, n2g_col_ref, n2g_row_ref, eps_ref,
                gw_ref, gb_ref, out_ref):
    x = x_ref[...]
    h = aggA_ref[...] + aggB_ref[...] + (1.0 + eps_ref[0, 0]) * x

    gid_cols = lax.broadcasted_iota(jnp.int32, (N_NODES, D), 1)
    oh = (gid_cols == n2g_col_ref[...]).astype(jnp.float32)
    gid_rows = lax.broadcasted_iota(jnp.int32, (D, N_NODES), 0)
    ohT = (gid_rows == n2g_row_ref[...]).astype(jnp.float32)

    cnt = jnp.sum(ohT, axis=1, keepdims=True)
    norm = jnp.maximum(cnt, 1.0) * jnp.float32(D)

    s1 = jnp.dot(ohT, h, preferred_element_type=jnp.float32)
    mean_g = jnp.sum(s1, axis=1, keepdims=True) / norm
    mean_n = jnp.dot(oh, mean_g, preferred_element_type=jnp.float32)
    xc = h - mean_n
    s2 = jnp.dot(ohT, xc * xc, preferred_element_type=jnp.float32)
    var_g = jnp.sum(s2, axis=1, keepdims=True) / norm
    rstd_g = lax.rsqrt(var_g + jnp.float32(LN_EPS))
    rstd_n = jnp.dot(oh, rstd_g, preferred_element_type=jnp.float32)

    y = xc * rstd_n * gw_ref[...] + gb_ref[...] + x
    out_ref[...] = jnp.maximum(y, 0.0)


def _final_call(aggA, aggB, x, n2g_col, n2g_row, eps, gw, gb):
    return pl.pallas_call(
        _final_body,
        out_shape=jax.ShapeDtypeStruct((N_NODES, D), jnp.float32),
    )(aggA, aggB, x, n2g_col, n2g_row, eps, gw, gb)


def kernel(x, edge_index, edge_attr, node2graph,
           We1, be1, Wm1, bm1, eps1, gn1_w, gn1_b,
           We2, be2, Wm2, bm2, eps2, gn2_w, gn2_b):
    n_pad_edges = E_PAD - N_EDGES
    src = jnp.concatenate(
        [edge_index[0], jnp.zeros((n_pad_edges,), jnp.int32)]
    ).reshape(NW, NCH, CH)
    dst = jnp.concatenate(
        [edge_index[1], jnp.full((n_pad_edges,), N_NODES, jnp.int32)]
    ).reshape(NW, NCH, CH)
    ea = jnp.concatenate(
        [edge_attr, jnp.zeros((n_pad_edges, ED), jnp.float32)], axis=0
    )

    x_pad = jnp.concatenate(
        [x, jnp.zeros((N_PAD - N_NODES, D), jnp.float32)], axis=0
    )
    xg = _make_sc_gather()(x_pad, src).reshape(E_PAD, D)
    msg = _msg_call(ea, xg, We2, be2.reshape(1, D), Wm2, bm2.reshape(1, D))
    agg2 = _make_sc_scatter()(msg.reshape(NW, NCH, CH, D), dst)
    out = _final_call(
        agg2[0, :N_NODES], agg2[1, :N_NODES], x,
        node2graph.reshape(N_NODES, 1), node2graph.reshape(1, N_NODES),
        eps2.reshape(1, 1), gn2_w.reshape(1, D), gn2_b.reshape(1, D),
    )
    return out
